# Initial kernel scaffold; baseline (speedup 1.0000x reference)
import jax
import jax.numpy as jnp
from jax.experimental import pallas as pl


def kernel(x, edge_index, W1, b1, W2, b2):
    raise NotImplementedError("write your pallas kernel here")


# speedup vs baseline: 63.0936x; 63.0936x over previous
import functools

import jax
import jax.numpy as jnp
from jax import lax
from jax.experimental import pallas as pl
from jax.experimental.pallas import tpu as pltpu
from jax.experimental.pallas import tpu_sc as plsc

_N = 10000
_E = 640000
_H = 16
_NC, _NS = 2, 16
_NW = _NC * _NS
_EPW = _E // _NW
_CH = 2000
_NCHUNK = _EPW // _CH
_NPAD = 10240
_RPT = _NPAD // _NS

_MESH = plsc.VectorSubcoreMesh(core_axis_name="c", subcore_axis_name="s")


@functools.partial(
    pl.kernel,
    out_type=(
        jax.ShapeDtypeStruct((_NPAD,), jnp.float32),
        jax.ShapeDtypeStruct((_NPAD,), jnp.float32),
    ),
    mesh=_MESH,
    scratch_types=[
        pltpu.VMEM((_CH,), jnp.int32),
        pltpu.VMEM((_CH,), jnp.float32),
        pltpu.VMEM_SHARED((_NPAD,), jnp.float32),
    ],
    compiler_params=pltpu.CompilerParams(use_tc_tiling_on_sc=False),
)
def _sc_degree(dst_hbm, zeros_hbm, ones_hbm, d0_hbm, d1_hbm, idx_v, ones_v, acc_sh):
    c = lax.axis_index("c")
    s = lax.axis_index("s")
    wid = s * _NC + c
    pltpu.sync_copy(zeros_hbm.at[pl.ds(s * _RPT, _RPT)], acc_sh.at[pl.ds(s * _RPT, _RPT)])
    pltpu.sync_copy(ones_hbm, ones_v)
    plsc.subcore_barrier()
    base = wid * _EPW
    for k in range(_NCHUNK):
        pltpu.sync_copy(dst_hbm.at[pl.ds(base + k * _CH, _CH)], idx_v)
        pltpu.sync_copy(ones_v, acc_sh.at[idx_v], add=True)
    plsc.subcore_barrier()

    @pl.when(c == 0)
    def _():
        pltpu.sync_copy(acc_sh.at[pl.ds(s * _RPT, _RPT)], d0_hbm.at[pl.ds(s * _RPT, _RPT)])

    @pl.when(c == 1)
    def _():
        pltpu.sync_copy(acc_sh.at[pl.ds(s * _RPT, _RPT)], d1_hbm.at[pl.ds(s * _RPT, _RPT)])


@functools.partial(
    pl.kernel,
    out_type=(
        jax.ShapeDtypeStruct((_NPAD, _H), jnp.float32),
        jax.ShapeDtypeStruct((_NPAD, _H), jnp.float32),
    ),
    mesh=_MESH,
    scratch_types=[
        pltpu.VMEM((_CH,), jnp.int32),
        pltpu.VMEM((_CH,), jnp.int32),
        pltpu.VMEM((_CH, _H), jnp.float32),
        pltpu.VMEM_SHARED((_NPAD, _H), jnp.float32),
        pltpu.SemaphoreType.DMA,
    ],
    compiler_params=pltpu.CompilerParams(use_tc_tiling_on_sc=False),
)
def _sc_edge(t_hbm, src_hbm, dst_hbm, p0_hbm, p1_hbm, sidx_v, didx_v, rows_v, acc_sh, sem):
    c = lax.axis_index("c")
    s = lax.axis_index("s")
    wid = s * _NC + c
    pltpu.sync_copy(t_hbm.at[pl.ds(s * _RPT, _RPT)], acc_sh.at[pl.ds(s * _RPT, _RPT)])
    plsc.subcore_barrier()
    base = wid * _EPW
    for k in range(_NCHUNK):
        pltpu.sync_copy(src_hbm.at[pl.ds(base + k * _CH, _CH)], sidx_v)
        pltpu.sync_copy(dst_hbm.at[pl.ds(base + k * _CH, _CH)], didx_v)
        pltpu.async_copy(t_hbm.at[sidx_v], rows_v, sem).wait()
        pltpu.sync_copy(rows_v, acc_sh.at[didx_v], add=True)
    plsc.subcore_barrier()

    @pl.when(c == 0)
    def _():
        pltpu.sync_copy(acc_sh.at[pl.ds(s * _RPT, _RPT)], p0_hbm.at[pl.ds(s * _RPT, _RPT)])

    @pl.when(c == 1)
    def _():
        pltpu.sync_copy(acc_sh.at[pl.ds(s * _RPT, _RPT)], p1_hbm.at[pl.ds(s * _RPT, _RPT)])


_BLK = 1000
_BLKP = 1024


def _mm_scale_body(d0_ref, d1_ref, x_ref, w_ref, o_ref):
    dinv = lax.rsqrt(d0_ref[...] + d1_ref[...] + 1.0)
    h = jnp.dot(x_ref[...], w_ref[...], preferred_element_type=jnp.float32)
    o_ref[...] = h * dinv


def _tc_mm_scale(d0, d1, x, W1):
    f_in = x.shape[1]
    return pl.pallas_call(
        _mm_scale_body,
        grid=(_N // _BLK,),
        in_specs=[
            pl.BlockSpec((_BLK, 1), lambda i: (i, 0)),
            pl.BlockSpec((_BLK, 1), lambda i: (i, 0)),
            pl.BlockSpec((_BLK, f_in), lambda i: (i, 0)),
            pl.BlockSpec((f_in, _H), lambda i: (0, 0)),
        ],
        out_specs=pl.BlockSpec((_BLK, _H), lambda i: (i, 0)),
        out_shape=jax.ShapeDtypeStruct((_N, _H), jnp.float32),
    )(d0, d1, x, W1)


def _layer2_body(d0_ref, d1_ref, p0_ref, p1_ref, t1_ref, b1_ref, w2_ref, o_ref):
    dinv = lax.rsqrt(d0_ref[...] + d1_ref[...] + 1.0)
    agg = p0_ref[...] + p1_ref[...] - t1_ref[...]
    h1 = jnp.maximum(dinv * agg + b1_ref[...], 0.0)
    h2 = jnp.dot(h1, w2_ref[...], preferred_element_type=jnp.float32)
    o_ref[...] = dinv * h2


def _tc_layer2(d0, d1, p0, p1, t1, b1r, W2p):
    return pl.pallas_call(
        _layer2_body,
        grid=(_NPAD // _BLKP,),
        in_specs=[
            pl.BlockSpec((_BLKP, 1), lambda i: (i, 0)),
            pl.BlockSpec((_BLKP, 1), lambda i: (i, 0)),
            pl.BlockSpec((_BLKP, _H), lambda i: (i, 0)),
            pl.BlockSpec((_BLKP, _H), lambda i: (i, 0)),
            pl.BlockSpec((_BLKP, _H), lambda i: (i, 0)),
            pl.BlockSpec((1, _H), lambda i: (0, 0)),
            pl.BlockSpec((_H, _H), lambda i: (0, 0)),
        ],
        out_specs=pl.BlockSpec((_BLKP, _H), lambda i: (i, 0)),
        out_shape=jax.ShapeDtypeStruct((_NPAD, _H), jnp.float32),
    )(d0, d1, p0, p1, t1, b1r, W2p)


def _final_body(d0_ref, d1_ref, q0_ref, q1_ref, t2_ref, b2_ref, o_ref):
    dinv = lax.rsqrt(d0_ref[...] + d1_ref[...] + 1.0)
    z = dinv * (q0_ref[...] + q1_ref[...] - t2_ref[...]) + b2_ref[...]
    col = lax.broadcasted_iota(jnp.int32, z.shape, 1)
    valid = col < 7
    zm = jnp.where(valid, z, -jnp.inf)
    m = jnp.max(zm, axis=1, keepdims=True)
    e = jnp.where(valid, jnp.exp(z - m), 0.0)
    ssum = jnp.sum(e, axis=1, keepdims=True)
    o_ref[...] = z - m - jnp.log(ssum)


def _tc_final(d0, d1, q0, q1, t2, b2p):
    return pl.pallas_call(
        _final_body,
        grid=(_NPAD // _BLKP,),
        in_specs=[
            pl.BlockSpec((_BLKP, 1), lambda i: (i, 0)),
            pl.BlockSpec((_BLKP, 1), lambda i: (i, 0)),
            pl.BlockSpec((_BLKP, _H), lambda i: (i, 0)),
            pl.BlockSpec((_BLKP, _H), lambda i: (i, 0)),
            pl.BlockSpec((_BLKP, _H), lambda i: (i, 0)),
            pl.BlockSpec((1, _H), lambda i: (0, 0)),
        ],
        out_specs=pl.BlockSpec((_BLKP, _H), lambda i: (i, 0)),
        out_shape=jax.ShapeDtypeStruct((_NPAD, _H), jnp.float32),
    )(d0, d1, q0, q1, t2, b2p)


def kernel(x, edge_index, W1, b1, W2, b2):
    src = edge_index[0]
    dst = edge_index[1]
    zeros_n = jnp.zeros((_NPAD,), jnp.float32)
    ones_ch = jnp.ones((_CH,), jnp.float32)

    d0p, d1p = _sc_degree(dst, zeros_n, ones_ch)
    d0 = d0p.reshape(_NPAD, 1)
    d1 = d1p.reshape(_NPAD, 1)

    t1 = _tc_mm_scale(d0[:_N], d1[:_N], x, W1)
    t1p = jnp.pad(t1, ((0, _NPAD - _N), (0, 0)))
    p0, p1 = _sc_edge(t1p, src, dst)

    b1r = b1.reshape(1, _H)
    W2p = jnp.zeros((_H, _H), jnp.float32).at[:, : W2.shape[1]].set(W2)
    t2 = _tc_layer2(d0, d1, p0, p1, t1p, b1r, W2p)

    q0, q1 = _sc_edge(t2, src, dst)

    b2p = jnp.zeros((1, _H), jnp.float32).at[0, : b2.shape[0]].set(b2)
    out = _tc_final(d0, d1, q0, q1, t2, b2p)
    return out[:_N, :7]
